# baseline (speedup 1.0000x reference)
import functools

import jax
import jax.numpy as jnp
from jax import lax
from jax.experimental import pallas as pl
from jax.experimental.pallas import tpu as pltpu
import jax.experimental.pallas.tpu_sc as plsc

N = 10000
E = 320000
F_IN = 128
HIDDEN = 64
HEADS = 8
N_CLASSES = 40

NW = 32
RPW = 320
N_PAD = NW * RPW
CHUNK = 64
MAXSPAN = 12288
EP = E + N
SRC_PAD = EP + MAXSPAN + 64
RP_PAD = N_PAD + 24
C1 = HEADS * HIDDEN
C2 = 64
F32 = jnp.float32


def _k1_body(x_ref, w_ref, a_ref, h_ref, ao_ref):
    h = jnp.dot(x_ref[...], w_ref[...], preferred_element_type=F32,
                precision=lax.Precision.HIGHEST)
    h_ref[...] = h
    ao_ref[...] = jnp.dot(h, a_ref[...], preferred_element_type=F32,
                          precision=lax.Precision.HIGHEST)


def _k2_body(x_ref, b_ref, w_ref, o_ref):
    v = x_ref[...] + b_ref[...]
    o = jnp.where(v > 0, v, jnp.exp(jnp.minimum(v, 0.0)) - 1.0)
    o_ref[...] = jnp.dot(o, w_ref[...], preferred_element_type=F32,
                         precision=lax.Precision.HIGHEST)


def _k3_body(x_ref, b_ref, o_ref):
    x = x_ref[...] + b_ref[...]
    m = jnp.max(x, axis=-1, keepdims=True)
    s = jnp.log(jnp.sum(jnp.exp(x - m), axis=-1, keepdims=True))
    o_ref[...] = x - m - s


def _iota16():
    return lax.broadcasted_iota(jnp.int32, (16,), 0)


def _sc_agg_body(nfeat, nheads, chunk, rp_hbm, src_hbm, dst_hbm, prev_hbm, a_hbm,
                 h_hbm, out_hbm, rp_v, src_v, av_v, hb, ab, db, pv, pb, obuf,
                 sh, sa):
    fh = nfeat // nheads
    vh = fh // 16
    nv = nfeat // 16
    fused = nfeat == C2
    scol = 40 if fused else 0
    acol = 41 if fused else nheads

    cid = lax.axis_index("c")
    sid = lax.axis_index("s")
    wid = sid * 2 + cid
    d0 = pl.multiple_of(wid * RPW, RPW)

    pltpu.sync_copy(rp_hbm.at[pl.ds(0, 48)], rp_v)
    pltpu.sync_copy(a_hbm.at[pl.ds(d0, RPW)], av_v)
    e0 = rp_v[pl.ds(wid, 16)][0]
    e1 = rp_v[pl.ds(wid + 1, 16)][0]
    e0a = pl.multiple_of((e0 // 8) * 8, 8)
    pltpu.sync_copy(src_hbm.at[pl.ds(e0a, MAXSPAN)], src_v)
    nchunks = jnp.minimum((e1 - e0a + chunk - 1) // chunk, MAXSPAN // chunk)

    def issue(c, par):
        @pl.when(c < nchunks)
        def _():
            idx = src_v.at[pl.ds(c * chunk, chunk)]
            pltpu.async_copy(h_hbm.at[idx], hb[par], sh[par])
            if not fused:
                pltpu.async_copy(a_hbm.at[idx], ab[par], sa[par])
            off = pl.multiple_of(e0a + c * chunk, 8)
            pltpu.async_copy(dst_hbm.at[pl.ds(off, chunk)], db[par], sa[par])
            pltpu.async_copy(prev_hbm.at[pl.ds(off, chunk)], pv[par], sa[par])

    issue(0, 0)
    issue(1, 1)

    zacc = tuple(jnp.zeros((16,), F32) for _ in range(nv))
    zds = tuple(jnp.zeros((16,), F32) for _ in range(nheads))

    def finalize(d, accs, dss, flush_all):
        r = d - d0
        rloc = lax.rem(r, 32)
        for h in range(nheads):
            winv = 1.0 / (dss[h] + 1e-16)
            for v in range(vh):
                obuf[rloc, pl.ds((h * vh + v) * 16, 16)] = accs[h * vh + v] * winv

        @pl.when(jnp.logical_or(rloc == 31, flush_all))
        def _():
            pltpu.sync_copy(obuf, out_hbm.at[pl.ds(pl.multiple_of(d - rloc, 32), 32)])

    def process_chunk(c, par, carry):
        @pl.when(c < nchunks)
        def _():
            pltpu.make_async_copy(h_hbm.at[pl.ds(0, chunk)], hb[par], sh[par]).wait()
            if not fused:
                pltpu.make_async_copy(a_hbm.at[pl.ds(0, chunk)], ab[par], sa[par]).wait()
            pltpu.make_async_copy(dst_hbm.at[pl.ds(0, chunk)], db[par], sa[par]).wait()
            pltpu.make_async_copy(prev_hbm.at[pl.ds(0, chunk)], pv[par], sa[par]).wait()

        srcrows = hb[par] if fused else ab[par]
        for j in range(chunk // 16):
            e16 = _iota16() + j * 16
            dst16 = db[par][pl.ds(j * 16, 16)]
            prev16 = pv[par][pl.ds(j * 16, 16)]
            dloc16 = jnp.clip(dst16 - d0, 0, RPW - 1)
            bnd = (dst16 != prev16).astype(F32)
            plsc.store_scatter(pb[par], [e16, jnp.full((16,), 8, jnp.int32)], bnd)
            plsc.store_scatter(pb[par], [e16, jnp.full((16,), 9, jnp.int32)],
                               dst16.astype(F32))
            for h in range(nheads):
                asrc = plsc.load_gather(srcrows, [e16, jnp.full((16,), scol + h, jnp.int32)])
                adst = plsc.load_gather(av_v, [dloc16, jnp.full((16,), acol + h, jnp.int32)])
                a = asrc + adst
                p = jnp.exp(jnp.maximum(a, 0.2 * a))
                plsc.store_scatter(pb[par], [e16, jnp.full((16,), h, jnp.int32)], p)

        cbase = e0a + c * chunk
        lo = jnp.maximum(0, e0 - cbase)
        hi = jnp.minimum(chunk, e1 - cbase)

        def edge_body(e, ecarry):
            d, accs, dss = ecarry
            prow = pb[par][e, pl.ds(0, 16)]
            boundary = prow[8] > 0.5

            @pl.when(boundary)
            def _():
                finalize(d, accs, dss, jnp.bool_(False))

            d = jnp.where(boundary, prow[9].astype(jnp.int32), d)
            keep = jnp.where(boundary, jnp.float32(0.0), jnp.float32(1.0))

            ps = [prow[h] for h in range(nheads)]
            dss = tuple(dss[h] * keep + ps[h] for h in range(nheads))
            accs = tuple(
                accs[v] * keep + ps[v // vh] * hb[par][e, pl.ds(v * 16, 16)]
                for v in range(nv))
            return d, accs, dss

        carry = lax.fori_loop(lo, hi, edge_body, carry)

        issue(c + 2, par)
        return carry

    carry0 = (d0, zacc, zds)
    npairs = (nchunks + 1) // 2

    def pair_body(pi, carry):
        for par in range(2):
            carry = process_chunk(pi * 2 + par, par, carry)
        return carry

    d, accs, dss = lax.fori_loop(0, npairs, pair_body, carry0)
    finalize(d, accs, dss, jnp.bool_(True))


def _make_sc_agg(nfeat, nheads, awidth, chunk):
    mesh = plsc.VectorSubcoreMesh(core_axis_name="c", subcore_axis_name="s")

    @functools.partial(
        pl.kernel,
        out_type=jax.ShapeDtypeStruct((N_PAD, nfeat), F32),
        mesh=mesh,
        compiler_params=pltpu.CompilerParams(needs_layout_passes=False,
                                             use_tc_tiling_on_sc=False),
        scratch_types=dict(
            rp_v=pltpu.VMEM((48,), jnp.int32),
            src_v=pltpu.VMEM((MAXSPAN,), jnp.int32),
            av_v=pltpu.VMEM((RPW, awidth), F32),
            hb=[pltpu.VMEM((chunk, nfeat), F32)] * 2,
            ab=[pltpu.VMEM((chunk, awidth), F32)] * 2,
            db=[pltpu.VMEM((chunk,), jnp.int32)] * 2,
            pv=[pltpu.VMEM((chunk,), jnp.int32)] * 2,
            pb=[pltpu.VMEM((chunk, 16), F32)] * 2,
            obuf=pltpu.VMEM((32, nfeat), F32),
            sh=[pltpu.SemaphoreType.DMA] * 2,
            sa=[pltpu.SemaphoreType.DMA] * 2,
        ),
    )
    def agg(rp_hbm, src_hbm, dst_hbm, prev_hbm, a_hbm, h_hbm, out_hbm, *,
            rp_v, src_v, av_v, hb, ab, db, pv, pb, obuf, sh, sa):
        _sc_agg_body(nfeat, nheads, chunk, rp_hbm, src_hbm, dst_hbm, prev_hbm,
                     a_hbm, h_hbm, out_hbm, rp_v, src_v, av_v, hb, ab, db,
                     pv, pb, obuf, sh, sa)

    return agg


_agg1 = _make_sc_agg(C1, HEADS, 16, 64)
_agg2 = _make_sc_agg(C2, 1, C2, 256)


def kernel(x, edge_index, W1, att_src1, att_dst1, b1, W2, att_src2, att_dst2, b2):
    loop = jnp.arange(N, dtype=jnp.int32)
    src = jnp.concatenate([edge_index[0], loop])
    dst = jnp.concatenate([edge_index[1], loop])
    dst_s, src_s = lax.sort((dst, src), num_keys=1)
    rp32 = jnp.searchsorted(dst_s, jnp.arange(0, N_PAD + 1, RPW,
                                              dtype=jnp.int32)).astype(jnp.int32)
    rp32 = jnp.concatenate([rp32, jnp.full((15,), EP, jnp.int32)])
    src_sp = jnp.pad(src_s, (0, SRC_PAD - EP))
    dst_sp = jnp.pad(dst_s, (0, SRC_PAD - EP))
    prev_sp = jnp.concatenate([jnp.full((1,), -1, jnp.int32), dst_sp[:-1]])

    att_s1 = att_src1.reshape(HEADS, HIDDEN)
    att_d1 = att_dst1.reshape(HEADS, HIDDEN)
    eye = jnp.eye(HEADS, dtype=F32)
    A1 = jnp.concatenate(
        [(att_s1[:, :, None] * eye[:, None, :]).reshape(C1, HEADS),
         (att_d1[:, :, None] * eye[:, None, :]).reshape(C1, HEADS)], axis=1)

    xp = jnp.pad(x, ((0, N_PAD - N), (0, 0)))
    h1, a1 = pl.pallas_call(
        _k1_body,
        out_shape=(jax.ShapeDtypeStruct((N_PAD, C1), F32),
                   jax.ShapeDtypeStruct((N_PAD, 16), F32)),
        grid=(NW,),
        in_specs=[
            pl.BlockSpec((RPW, F_IN), lambda i: (i, 0)),
            pl.BlockSpec((F_IN, C1), lambda i: (0, 0)),
            pl.BlockSpec((C1, 16), lambda i: (0, 0)),
        ],
        out_specs=(pl.BlockSpec((RPW, C1), lambda i: (i, 0)),
                   pl.BlockSpec((RPW, 16), lambda i: (i, 0))),
    )(xp, W1, A1)

    out1 = _agg1(rp32, src_sp, dst_sp, prev_sp, a1, h1)

    W2p = jnp.concatenate(
        [W2, W2 @ att_src2.reshape(N_CLASSES, 1), W2 @ att_dst2.reshape(N_CLASSES, 1),
         jnp.zeros((C1, C2 - N_CLASSES - 2), F32)], axis=1)

    h2p = pl.pallas_call(
        _k2_body,
        out_shape=jax.ShapeDtypeStruct((N_PAD, C2), F32),
        grid=(NW,),
        in_specs=[
            pl.BlockSpec((RPW, C1), lambda i: (i, 0)),
            pl.BlockSpec((C1,), lambda i: (0,)),
            pl.BlockSpec((C1, C2), lambda i: (0, 0)),
        ],
        out_specs=pl.BlockSpec((RPW, C2), lambda i: (i, 0)),
    )(out1, b1, W2p)

    out2 = _agg2(rp32, src_sp, dst_sp, prev_sp, h2p, h2p)

    return pl.pallas_call(
        _k3_body,
        out_shape=jax.ShapeDtypeStruct((N, N_CLASSES), F32),
        grid=(10,),
        in_specs=[
            pl.BlockSpec((N // 10, N_CLASSES), lambda i: (i, 0)),
            pl.BlockSpec((N_CLASSES,), lambda i: (0,)),
        ],
        out_specs=pl.BlockSpec((N // 10, N_CLASSES), lambda i: (i, 0)),
    )(out2[:N, :N_CLASSES], b2)


# speedup vs baseline: 1.0147x; 1.0147x over previous
import functools

import jax
import jax.numpy as jnp
from jax import lax
from jax.experimental import pallas as pl
from jax.experimental.pallas import tpu as pltpu
import jax.experimental.pallas.tpu_sc as plsc

N = 10000
E = 320000
F_IN = 128
HIDDEN = 64
HEADS = 8
N_CLASSES = 40

NW = 32
RPW = 320
N_PAD = NW * RPW
CHUNK = 64
MAXSPAN = 12288
EP = E + N
SRC_PAD = EP + MAXSPAN + 64
RP_PAD = N_PAD + 24
C1 = HEADS * HIDDEN
C2 = 64
F32 = jnp.float32


def _k1_body(x_ref, w_ref, a_ref, h_ref, ao_ref):
    h = jnp.dot(x_ref[...], w_ref[...], preferred_element_type=F32,
                precision=lax.Precision.HIGHEST)
    h_ref[...] = h
    ao_ref[...] = jnp.dot(h, a_ref[...], preferred_element_type=F32,
                          precision=lax.Precision.HIGHEST)


def _k2_body(x_ref, b_ref, w_ref, o_ref):
    v = x_ref[...] + b_ref[...]
    o = jnp.where(v > 0, v, jnp.exp(jnp.minimum(v, 0.0)) - 1.0)
    o_ref[...] = jnp.dot(o, w_ref[...], preferred_element_type=F32,
                         precision=lax.Precision.HIGHEST)


def _k3_body(x_ref, b_ref, o_ref):
    x = x_ref[...] + b_ref[...]
    m = jnp.max(x, axis=-1, keepdims=True)
    s = jnp.log(jnp.sum(jnp.exp(x - m), axis=-1, keepdims=True))
    o_ref[...] = x - m - s


def _iota16():
    return lax.broadcasted_iota(jnp.int32, (16,), 0)


def _sc_agg_body(nfeat, nheads, chunk, rp_hbm, src_hbm, dst_hbm, prev_hbm, a_hbm,
                 h_hbm, out_hbm, rp_v, src_v, av_v, hb, ab, db, pv, pb, obuf,
                 sh, sa):
    fh = nfeat // nheads
    vh = fh // 16
    nv = nfeat // 16
    fused = nfeat == C2
    scol = 40 if fused else 0
    acol = 41 if fused else nheads

    cid = lax.axis_index("c")
    sid = lax.axis_index("s")
    wid = sid * 2 + cid
    d0 = pl.multiple_of(wid * RPW, RPW)

    pltpu.sync_copy(rp_hbm.at[pl.ds(0, 48)], rp_v)
    pltpu.sync_copy(a_hbm.at[pl.ds(d0, RPW)], av_v)
    e0 = rp_v[pl.ds(wid, 16)][0]
    e1 = rp_v[pl.ds(wid + 1, 16)][0]
    e0a = pl.multiple_of((e0 // 8) * 8, 8)
    pltpu.sync_copy(src_hbm.at[pl.ds(e0a, MAXSPAN)], src_v)
    nchunks = jnp.minimum((e1 - e0a + chunk - 1) // chunk, MAXSPAN // chunk)

    def issue(c, par):
        @pl.when(c < nchunks)
        def _():
            idx = src_v.at[pl.ds(c * chunk, chunk)]
            pltpu.async_copy(h_hbm.at[idx], hb[par], sh[par])
            if not fused:
                pltpu.async_copy(a_hbm.at[idx], ab[par], sa[par])
            off = pl.multiple_of(e0a + c * chunk, 8)
            pltpu.async_copy(dst_hbm.at[pl.ds(off, chunk)], db[par], sa[par])
            pltpu.async_copy(prev_hbm.at[pl.ds(off, chunk)], pv[par], sa[par])

    issue(0, 0)
    issue(1, 1)

    zacc = tuple(jnp.zeros((16,), F32) for _ in range(nv))
    zds = tuple(jnp.zeros((16,), F32) for _ in range(nheads))

    def finalize(d, accs, dss, flush_all):
        r = d - d0
        rloc = lax.rem(r, 32)
        for h in range(nheads):
            winv = 1.0 / (dss[h] + 1e-16)
            for v in range(vh):
                obuf[rloc, pl.ds((h * vh + v) * 16, 16)] = accs[h * vh + v] * winv

        @pl.when(jnp.logical_or(rloc == 31, flush_all))
        def _():
            pltpu.sync_copy(obuf, out_hbm.at[pl.ds(pl.multiple_of(d - rloc, 32), 32)])

    def process_chunk(c, par, carry):
        @pl.when(c < nchunks)
        def _():
            pltpu.make_async_copy(h_hbm.at[pl.ds(0, chunk)], hb[par], sh[par]).wait()
            if not fused:
                pltpu.make_async_copy(a_hbm.at[pl.ds(0, chunk)], ab[par], sa[par]).wait()
            pltpu.make_async_copy(dst_hbm.at[pl.ds(0, chunk)], db[par], sa[par]).wait()
            pltpu.make_async_copy(prev_hbm.at[pl.ds(0, chunk)], pv[par], sa[par]).wait()

        srcrows = hb[par] if fused else ab[par]
        for j in range(chunk // 16):
            e16 = _iota16() + j * 16
            dst16 = db[par][pl.ds(j * 16, 16)]
            prev16 = pv[par][pl.ds(j * 16, 16)]
            dloc16 = jnp.clip(dst16 - d0, 0, RPW - 1)
            bnd = (dst16 != prev16).astype(F32)
            plsc.store_scatter(pb[par], [e16, jnp.full((16,), 8, jnp.int32)], bnd)
            plsc.store_scatter(pb[par], [e16, jnp.full((16,), 9, jnp.int32)],
                               dst16.astype(F32))
            for h in range(nheads):
                asrc = plsc.load_gather(srcrows, [e16, jnp.full((16,), scol + h, jnp.int32)])
                adst = plsc.load_gather(av_v, [dloc16, jnp.full((16,), acol + h, jnp.int32)])
                a = asrc + adst
                p = jnp.exp(jnp.maximum(a, 0.2 * a))
                plsc.store_scatter(pb[par], [e16, jnp.full((16,), h, jnp.int32)], p)

        cbase = e0a + c * chunk
        lo = jnp.maximum(0, e0 - cbase)
        hi = jnp.minimum(chunk, e1 - cbase)

        def edge_body(e, ecarry):
            d, accs, dss = ecarry
            prow = pb[par][e, pl.ds(0, 16)]
            boundary = prow[8] > 0.5

            @pl.when(boundary)
            def _():
                finalize(d, accs, dss, jnp.bool_(False))

            d = jnp.where(boundary, prow[9].astype(jnp.int32), d)
            keep = jnp.where(boundary, jnp.float32(0.0), jnp.float32(1.0))

            ps = [prow[h] for h in range(nheads)]
            dss = tuple(dss[h] * keep + ps[h] for h in range(nheads))
            accs = tuple(
                accs[v] * keep + ps[v // vh] * hb[par][e, pl.ds(v * 16, 16)]
                for v in range(nv))
            return d, accs, dss

        carry = lax.fori_loop(lo, hi, edge_body, carry)

        issue(c + 2, par)
        return carry

    carry0 = (d0, zacc, zds)
    npairs = (nchunks + 1) // 2

    def pair_body(pi, carry):
        for par in range(2):
            carry = process_chunk(pi * 2 + par, par, carry)
        return carry

    d, accs, dss = lax.fori_loop(0, npairs, pair_body, carry0)
    finalize(d, accs, dss, jnp.bool_(True))


def _make_sc_agg(nfeat, nheads, awidth, chunk):
    mesh = plsc.VectorSubcoreMesh(core_axis_name="c", subcore_axis_name="s")

    @functools.partial(
        pl.kernel,
        out_type=jax.ShapeDtypeStruct((N_PAD, nfeat), F32),
        mesh=mesh,
        compiler_params=pltpu.CompilerParams(needs_layout_passes=False,
                                             use_tc_tiling_on_sc=False),
        scratch_types=dict(
            rp_v=pltpu.VMEM((48,), jnp.int32),
            src_v=pltpu.VMEM((MAXSPAN,), jnp.int32),
            av_v=pltpu.VMEM((RPW, awidth), F32),
            hb=[pltpu.VMEM((chunk, nfeat), F32)] * 2,
            ab=[pltpu.VMEM((chunk, awidth), F32)] * 2,
            db=[pltpu.VMEM((chunk,), jnp.int32)] * 2,
            pv=[pltpu.VMEM((chunk,), jnp.int32)] * 2,
            pb=[pltpu.VMEM((chunk, 16), F32)] * 2,
            obuf=pltpu.VMEM((32, nfeat), F32),
            sh=[pltpu.SemaphoreType.DMA] * 2,
            sa=[pltpu.SemaphoreType.DMA] * 2,
        ),
    )
    def agg(rp_hbm, src_hbm, dst_hbm, prev_hbm, a_hbm, h_hbm, out_hbm, *,
            rp_v, src_v, av_v, hb, ab, db, pv, pb, obuf, sh, sa):
        _sc_agg_body(nfeat, nheads, chunk, rp_hbm, src_hbm, dst_hbm, prev_hbm,
                     a_hbm, h_hbm, out_hbm, rp_v, src_v, av_v, hb, ab, db,
                     pv, pb, obuf, sh, sa)

    return agg


_agg1 = _make_sc_agg(C1, HEADS, 16, 64)
_agg2 = _make_sc_agg(C2, 1, C2, 256)


def kernel(x, edge_index, W1, att_src1, att_dst1, b1, W2, att_src2, att_dst2, b2):
    loop = jnp.arange(N, dtype=jnp.int32)
    src = jnp.concatenate([edge_index[0], loop])
    dst = jnp.concatenate([edge_index[1], loop])
    packed = lax.sort((dst << 14) | src)
    dst_s = packed >> 14
    src_s = packed & jnp.int32(16383)
    rp32 = jnp.searchsorted(dst_s, jnp.arange(0, N_PAD + 1, RPW,
                                              dtype=jnp.int32)).astype(jnp.int32)
    rp32 = jnp.concatenate([rp32, jnp.full((15,), EP, jnp.int32)])
    src_sp = jnp.pad(src_s, (0, SRC_PAD - EP))
    dst_sp = jnp.pad(dst_s, (0, SRC_PAD - EP))
    prev_sp = jnp.concatenate([jnp.full((1,), -1, jnp.int32), dst_sp[:-1]])

    att_s1 = att_src1.reshape(HEADS, HIDDEN)
    att_d1 = att_dst1.reshape(HEADS, HIDDEN)
    eye = jnp.eye(HEADS, dtype=F32)
    A1 = jnp.concatenate(
        [(att_s1[:, :, None] * eye[:, None, :]).reshape(C1, HEADS),
         (att_d1[:, :, None] * eye[:, None, :]).reshape(C1, HEADS)], axis=1)

    xp = jnp.pad(x, ((0, N_PAD - N), (0, 0)))
    h1, a1 = pl.pallas_call(
        _k1_body,
        out_shape=(jax.ShapeDtypeStruct((N_PAD, C1), F32),
                   jax.ShapeDtypeStruct((N_PAD, 16), F32)),
        grid=(NW,),
        in_specs=[
            pl.BlockSpec((RPW, F_IN), lambda i: (i, 0)),
            pl.BlockSpec((F_IN, C1), lambda i: (0, 0)),
            pl.BlockSpec((C1, 16), lambda i: (0, 0)),
        ],
        out_specs=(pl.BlockSpec((RPW, C1), lambda i: (i, 0)),
                   pl.BlockSpec((RPW, 16), lambda i: (i, 0))),
    )(xp, W1, A1)

    out1 = _agg1(rp32, src_sp, dst_sp, prev_sp, a1, h1)

    W2p = jnp.concatenate(
        [W2, W2 @ att_src2.reshape(N_CLASSES, 1), W2 @ att_dst2.reshape(N_CLASSES, 1),
         jnp.zeros((C1, C2 - N_CLASSES - 2), F32)], axis=1)

    h2p = pl.pallas_call(
        _k2_body,
        out_shape=jax.ShapeDtypeStruct((N_PAD, C2), F32),
        grid=(NW,),
        in_specs=[
            pl.BlockSpec((RPW, C1), lambda i: (i, 0)),
            pl.BlockSpec((C1,), lambda i: (0,)),
            pl.BlockSpec((C1, C2), lambda i: (0, 0)),
        ],
        out_specs=pl.BlockSpec((RPW, C2), lambda i: (i, 0)),
    )(out1, b1, W2p)

    out2 = _agg2(rp32, src_sp, dst_sp, prev_sp, h2p, h2p)

    return pl.pallas_call(
        _k3_body,
        out_shape=jax.ShapeDtypeStruct((N, N_CLASSES), F32),
        grid=(10,),
        in_specs=[
            pl.BlockSpec((N // 10, N_CLASSES), lambda i: (i, 0)),
            pl.BlockSpec((N_CLASSES,), lambda i: (0,)),
        ],
        out_specs=pl.BlockSpec((N // 10, N_CLASSES), lambda i: (i, 0)),
    )(out2[:N, :N_CLASSES], b2)


# speedup vs baseline: 1.0421x; 1.0270x over previous
import functools

import jax
import jax.numpy as jnp
from jax import lax
from jax.experimental import pallas as pl
from jax.experimental.pallas import tpu as pltpu
import jax.experimental.pallas.tpu_sc as plsc

N = 10000
E = 320000
F_IN = 128
HIDDEN = 64
HEADS = 8
N_CLASSES = 40

NW = 32
RPW = 320
N_PAD = NW * RPW
CHUNK = 64
MAXSPAN = 12288
EP = E + N
SRC_PAD = EP + MAXSPAN + 64
RP_PAD = N_PAD + 24
C1 = HEADS * HIDDEN
C2 = 64
F32 = jnp.float32


def _k1_body(x_ref, w_ref, a_ref, h_ref, ao_ref):
    h = jnp.dot(x_ref[...], w_ref[...], preferred_element_type=F32,
                precision=lax.Precision.HIGHEST)
    h_ref[...] = h
    ao_ref[...] = jnp.dot(h, a_ref[...], preferred_element_type=F32,
                          precision=lax.Precision.HIGHEST)


def _k2_body(x_ref, b_ref, w_ref, o_ref):
    v = x_ref[...] + b_ref[...]
    o = jnp.where(v > 0, v, jnp.exp(jnp.minimum(v, 0.0)) - 1.0)
    o_ref[...] = jnp.dot(o, w_ref[...], preferred_element_type=F32,
                         precision=lax.Precision.HIGHEST)


def _k3_body(x_ref, b_ref, o_ref):
    x = x_ref[...] + b_ref[...]
    m = jnp.max(x, axis=-1, keepdims=True)
    s = jnp.log(jnp.sum(jnp.exp(x - m), axis=-1, keepdims=True))
    o_ref[...] = x - m - s


def _iota16():
    return lax.broadcasted_iota(jnp.int32, (16,), 0)


def _sc_agg_body(nfeat, nheads, chunk, rp_hbm, src_hbm, dst_hbm, prev_hbm, a_hbm,
                 h_hbm, out_hbm, rp_v, src_v, av_v, hb, ab, db, pv, pb, obuf,
                 sh, sa):
    fh = nfeat // nheads
    vh = fh // 16
    nv = nfeat // 16
    fused = nfeat == C2
    scol = 40 if fused else 0
    acol = 41 if fused else nheads

    cid = lax.axis_index("c")
    sid = lax.axis_index("s")
    wid = sid * 2 + cid
    d0 = pl.multiple_of(wid * RPW, RPW)

    pltpu.sync_copy(rp_hbm.at[pl.ds(0, 48)], rp_v)
    pltpu.sync_copy(a_hbm.at[pl.ds(d0, RPW)], av_v)
    e0 = rp_v[pl.ds(wid, 16)][0]
    e1 = rp_v[pl.ds(wid + 1, 16)][0]
    e0a = pl.multiple_of((e0 // 8) * 8, 8)
    pltpu.sync_copy(src_hbm.at[pl.ds(e0a, MAXSPAN)], src_v)
    nchunks = jnp.minimum((e1 - e0a + chunk - 1) // chunk, MAXSPAN // chunk)

    def issue(c, par):
        @pl.when(c < nchunks)
        def _():
            idx = src_v.at[pl.ds(c * chunk, chunk)]
            pltpu.async_copy(h_hbm.at[idx], hb[par], sh[par])
            if not fused:
                pltpu.async_copy(a_hbm.at[idx], ab[par], sa[par])
            off = pl.multiple_of(e0a + c * chunk, 8)
            pltpu.async_copy(dst_hbm.at[pl.ds(off, chunk)], db[par], sa[par])
            pltpu.async_copy(prev_hbm.at[pl.ds(off, chunk)], pv[par], sa[par])

    issue(0, 0)
    issue(1, 1)

    zacc = tuple(jnp.zeros((16,), F32) for _ in range(nv))
    zds = tuple(jnp.zeros((16,), F32) for _ in range(nheads))

    def finalize(d, accs, dss, flush_all):
        r = d - d0
        rloc = lax.rem(r, 32)
        for h in range(nheads):
            winv = 1.0 / (dss[h] + 1e-16)
            for v in range(vh):
                obuf[rloc, pl.ds((h * vh + v) * 16, 16)] = accs[h * vh + v] * winv

        @pl.when(jnp.logical_or(rloc == 31, flush_all))
        def _():
            pltpu.sync_copy(obuf, out_hbm.at[pl.ds(pl.multiple_of(d - rloc, 32), 32)])

    def process_chunk(c, par, carry):
        @pl.when(c < nchunks)
        def _():
            pltpu.make_async_copy(h_hbm.at[pl.ds(0, chunk)], hb[par], sh[par]).wait()
            if not fused:
                pltpu.make_async_copy(a_hbm.at[pl.ds(0, chunk)], ab[par], sa[par]).wait()
            pltpu.make_async_copy(dst_hbm.at[pl.ds(0, chunk)], db[par], sa[par]).wait()
            pltpu.make_async_copy(prev_hbm.at[pl.ds(0, chunk)], pv[par], sa[par]).wait()

        srcrows = hb[par] if fused else ab[par]
        for j in range(chunk // 16):
            e16 = _iota16() + j * 16
            dst16 = db[par][pl.ds(j * 16, 16)]
            prev16 = pv[par][pl.ds(j * 16, 16)]
            dloc16 = jnp.clip(dst16 - d0, 0, RPW - 1)
            bnd = (dst16 != prev16).astype(F32)
            plsc.store_scatter(pb[par], [e16, jnp.full((16,), 8, jnp.int32)], bnd)
            plsc.store_scatter(pb[par], [e16, jnp.full((16,), 9, jnp.int32)],
                               dst16.astype(F32))
            for h in range(nheads):
                asrc = plsc.load_gather(srcrows, [e16, jnp.full((16,), scol + h, jnp.int32)])
                adst = plsc.load_gather(av_v, [dloc16, jnp.full((16,), acol + h, jnp.int32)])
                a = asrc + adst
                p = jnp.exp(jnp.maximum(a, 0.2 * a))
                plsc.store_scatter(pb[par], [e16, jnp.full((16,), h, jnp.int32)], p)

        cbase = e0a + c * chunk
        lo = jnp.maximum(0, e0 - cbase)
        hi = jnp.minimum(chunk, e1 - cbase)

        def one_edge(e, valid, ecarry):
            d, accs, dss = ecarry
            prow = pb[par][e, pl.ds(0, 16)]
            boundary = jnp.logical_and(prow[8] > 0.5, valid)

            @pl.when(boundary)
            def _():
                finalize(d, accs, dss, jnp.bool_(False))

            d = jnp.where(boundary, prow[9].astype(jnp.int32), d)
            keep = jnp.where(boundary, jnp.float32(0.0), jnp.float32(1.0))
            vf = jnp.where(valid, jnp.float32(1.0), jnp.float32(0.0))

            ps = [prow[h] * vf for h in range(nheads)]
            dss = tuple(dss[h] * keep + ps[h] for h in range(nheads))
            accs = tuple(
                accs[v] * keep + ps[v // vh] * hb[par][e, pl.ds(v * 16, 16)]
                for v in range(nv))
            return d, accs, dss

        def edge_pair(i, ecarry):
            e = lo + i * 2
            ecarry = one_edge(e, jnp.bool_(True), ecarry)
            return one_edge(jnp.minimum(e + 1, chunk - 1), e + 1 < hi, ecarry)

        carry = lax.fori_loop(0, (hi - lo + 1) // 2, edge_pair, carry)

        issue(c + 2, par)
        return carry

    carry0 = (d0, zacc, zds)
    npairs = (nchunks + 1) // 2

    def pair_body(pi, carry):
        for par in range(2):
            carry = process_chunk(pi * 2 + par, par, carry)
        return carry

    d, accs, dss = lax.fori_loop(0, npairs, pair_body, carry0)
    finalize(d, accs, dss, jnp.bool_(True))


def _make_sc_agg(nfeat, nheads, awidth, chunk):
    mesh = plsc.VectorSubcoreMesh(core_axis_name="c", subcore_axis_name="s")

    @functools.partial(
        pl.kernel,
        out_type=jax.ShapeDtypeStruct((N_PAD, nfeat), F32),
        mesh=mesh,
        compiler_params=pltpu.CompilerParams(needs_layout_passes=False,
                                             use_tc_tiling_on_sc=False),
        scratch_types=dict(
            rp_v=pltpu.VMEM((48,), jnp.int32),
            src_v=pltpu.VMEM((MAXSPAN,), jnp.int32),
            av_v=pltpu.VMEM((RPW, awidth), F32),
            hb=[pltpu.VMEM((chunk, nfeat), F32)] * 2,
            ab=[pltpu.VMEM((chunk, awidth), F32)] * 2,
            db=[pltpu.VMEM((chunk,), jnp.int32)] * 2,
            pv=[pltpu.VMEM((chunk,), jnp.int32)] * 2,
            pb=[pltpu.VMEM((chunk, 16), F32)] * 2,
            obuf=pltpu.VMEM((32, nfeat), F32),
            sh=[pltpu.SemaphoreType.DMA] * 2,
            sa=[pltpu.SemaphoreType.DMA] * 2,
        ),
    )
    def agg(rp_hbm, src_hbm, dst_hbm, prev_hbm, a_hbm, h_hbm, out_hbm, *,
            rp_v, src_v, av_v, hb, ab, db, pv, pb, obuf, sh, sa):
        _sc_agg_body(nfeat, nheads, chunk, rp_hbm, src_hbm, dst_hbm, prev_hbm,
                     a_hbm, h_hbm, out_hbm, rp_v, src_v, av_v, hb, ab, db,
                     pv, pb, obuf, sh, sa)

    return agg


_agg1 = _make_sc_agg(C1, HEADS, 16, 64)
_agg2 = _make_sc_agg(C2, 1, C2, 256)


def kernel(x, edge_index, W1, att_src1, att_dst1, b1, W2, att_src2, att_dst2, b2):
    loop = jnp.arange(N, dtype=jnp.int32)
    src = jnp.concatenate([edge_index[0], loop])
    dst = jnp.concatenate([edge_index[1], loop])
    packed = lax.sort((dst << 14) | src)
    dst_s = packed >> 14
    src_s = packed & jnp.int32(16383)
    rp32 = jnp.searchsorted(dst_s, jnp.arange(0, N_PAD + 1, RPW,
                                              dtype=jnp.int32)).astype(jnp.int32)
    rp32 = jnp.concatenate([rp32, jnp.full((15,), EP, jnp.int32)])
    src_sp = jnp.pad(src_s, (0, SRC_PAD - EP))
    dst_sp = jnp.pad(dst_s, (0, SRC_PAD - EP))
    prev_sp = jnp.concatenate([jnp.full((1,), -1, jnp.int32), dst_sp[:-1]])

    att_s1 = att_src1.reshape(HEADS, HIDDEN)
    att_d1 = att_dst1.reshape(HEADS, HIDDEN)
    eye = jnp.eye(HEADS, dtype=F32)
    A1 = jnp.concatenate(
        [(att_s1[:, :, None] * eye[:, None, :]).reshape(C1, HEADS),
         (att_d1[:, :, None] * eye[:, None, :]).reshape(C1, HEADS)], axis=1)

    xp = jnp.pad(x, ((0, N_PAD - N), (0, 0)))
    h1, a1 = pl.pallas_call(
        _k1_body,
        out_shape=(jax.ShapeDtypeStruct((N_PAD, C1), F32),
                   jax.ShapeDtypeStruct((N_PAD, 16), F32)),
        grid=(NW,),
        in_specs=[
            pl.BlockSpec((RPW, F_IN), lambda i: (i, 0)),
            pl.BlockSpec((F_IN, C1), lambda i: (0, 0)),
            pl.BlockSpec((C1, 16), lambda i: (0, 0)),
        ],
        out_specs=(pl.BlockSpec((RPW, C1), lambda i: (i, 0)),
                   pl.BlockSpec((RPW, 16), lambda i: (i, 0))),
    )(xp, W1, A1)

    out1 = _agg1(rp32, src_sp, dst_sp, prev_sp, a1, h1)

    W2p = jnp.concatenate(
        [W2, W2 @ att_src2.reshape(N_CLASSES, 1), W2 @ att_dst2.reshape(N_CLASSES, 1),
         jnp.zeros((C1, C2 - N_CLASSES - 2), F32)], axis=1)

    h2p = pl.pallas_call(
        _k2_body,
        out_shape=jax.ShapeDtypeStruct((N_PAD, C2), F32),
        grid=(NW,),
        in_specs=[
            pl.BlockSpec((RPW, C1), lambda i: (i, 0)),
            pl.BlockSpec((C1,), lambda i: (0,)),
            pl.BlockSpec((C1, C2), lambda i: (0, 0)),
        ],
        out_specs=pl.BlockSpec((RPW, C2), lambda i: (i, 0)),
    )(out1, b1, W2p)

    out2 = _agg2(rp32, src_sp, dst_sp, prev_sp, h2p, h2p)

    return pl.pallas_call(
        _k3_body,
        out_shape=jax.ShapeDtypeStruct((N, N_CLASSES), F32),
        grid=(10,),
        in_specs=[
            pl.BlockSpec((N // 10, N_CLASSES), lambda i: (i, 0)),
            pl.BlockSpec((N_CLASSES,), lambda i: (0,)),
        ],
        out_specs=pl.BlockSpec((N // 10, N_CLASSES), lambda i: (i, 0)),
    )(out2[:N, :N_CLASSES], b2)


# speedup vs baseline: 1.0510x; 1.0085x over previous
import functools

import jax
import jax.numpy as jnp
from jax import lax
from jax.experimental import pallas as pl
from jax.experimental.pallas import tpu as pltpu
import jax.experimental.pallas.tpu_sc as plsc

N = 10000
E = 320000
F_IN = 128
HIDDEN = 64
HEADS = 8
N_CLASSES = 40

NW = 32
RPW = 320
N_PAD = NW * RPW
CHUNK = 64
MAXSPAN = 12288
EP = E + N
SRC_PAD = EP + MAXSPAN + 64
RP_PAD = N_PAD + 24
C1 = HEADS * HIDDEN
C2 = 64
F32 = jnp.float32


def _k1_body(x_ref, w_ref, a_ref, h_ref, ao_ref):
    h = jnp.dot(x_ref[...], w_ref[...], preferred_element_type=F32,
                precision=lax.Precision.HIGHEST)
    h_ref[...] = h
    ao_ref[...] = jnp.dot(h, a_ref[...], preferred_element_type=F32,
                          precision=lax.Precision.HIGHEST)


def _k2_body(x_ref, b_ref, w_ref, o_ref):
    v = x_ref[...] + b_ref[...]
    o = jnp.where(v > 0, v, jnp.exp(jnp.minimum(v, 0.0)) - 1.0)
    o_ref[...] = jnp.dot(o, w_ref[...], preferred_element_type=F32,
                         precision=lax.Precision.HIGHEST)


def _k3_body(x_ref, b_ref, o_ref):
    x = x_ref[...] + b_ref[...]
    m = jnp.max(x, axis=-1, keepdims=True)
    s = jnp.log(jnp.sum(jnp.exp(x - m), axis=-1, keepdims=True))
    o_ref[...] = x - m - s


def _iota16():
    return lax.broadcasted_iota(jnp.int32, (16,), 0)


def _sc_agg_body(nfeat, nheads, chunk, rp_hbm, src_hbm, dst_hbm, prev_hbm, a_hbm,
                 h_hbm, out_hbm, rp_v, src_v, av_v, hb, ab, db, pv, pb, obuf,
                 sh, sa):
    fh = nfeat // nheads
    vh = fh // 16
    nv = nfeat // 16
    fused = nfeat == C2
    scol = 40 if fused else 0
    acol = 41 if fused else nheads

    cid = lax.axis_index("c")
    sid = lax.axis_index("s")
    wid = sid * 2 + cid
    d0 = pl.multiple_of(wid * RPW, RPW)

    pltpu.sync_copy(rp_hbm.at[pl.ds(0, 48)], rp_v)
    pltpu.sync_copy(a_hbm.at[pl.ds(d0, RPW)], av_v)
    e0 = rp_v[pl.ds(wid, 16)][0]
    e1 = rp_v[pl.ds(wid + 1, 16)][0]
    e0a = pl.multiple_of((e0 // 8) * 8, 8)
    pltpu.sync_copy(src_hbm.at[pl.ds(e0a, MAXSPAN)], src_v)
    nchunks = jnp.minimum((e1 - e0a + chunk - 1) // chunk, MAXSPAN // chunk)

    def issue(c, par):
        @pl.when(c < nchunks)
        def _():
            idx = src_v.at[pl.ds(c * chunk, chunk)]
            pltpu.async_copy(h_hbm.at[idx], hb[par], sh[par])
            if not fused:
                pltpu.async_copy(a_hbm.at[idx], ab[par], sa[par])
            off = pl.multiple_of(e0a + c * chunk, 8)
            pltpu.async_copy(dst_hbm.at[pl.ds(off, chunk)], db[par], sa[par])
            pltpu.async_copy(prev_hbm.at[pl.ds(off, chunk)], pv[par], sa[par])

    issue(0, 0)
    issue(1, 1)

    zacc = tuple(jnp.zeros((16,), F32) for _ in range(nv))
    zds = tuple(jnp.float32(0.0) for _ in range(nheads))

    def finalize(d, accs, dss, flush_all):
        r = d - d0
        rloc = lax.rem(r, 32)
        for h in range(nheads):
            winv = 1.0 / (jnp.full((16,), dss[h], F32) + 1e-16)
            for v in range(vh):
                obuf[rloc, pl.ds((h * vh + v) * 16, 16)] = accs[h * vh + v] * winv

        @pl.when(jnp.logical_or(rloc == 31, flush_all))
        def _():
            pltpu.sync_copy(obuf, out_hbm.at[pl.ds(pl.multiple_of(d - rloc, 32), 32)])

    def process_chunk(c, par, carry):
        @pl.when(c < nchunks)
        def _():
            pltpu.make_async_copy(h_hbm.at[pl.ds(0, chunk)], hb[par], sh[par]).wait()
            if not fused:
                pltpu.make_async_copy(a_hbm.at[pl.ds(0, chunk)], ab[par], sa[par]).wait()
            pltpu.make_async_copy(dst_hbm.at[pl.ds(0, chunk)], db[par], sa[par]).wait()
            pltpu.make_async_copy(prev_hbm.at[pl.ds(0, chunk)], pv[par], sa[par]).wait()

        srcrows = hb[par] if fused else ab[par]
        for j in range(chunk // 16):
            e16 = _iota16() + j * 16
            dst16 = db[par][pl.ds(j * 16, 16)]
            prev16 = pv[par][pl.ds(j * 16, 16)]
            dloc16 = jnp.clip(dst16 - d0, 0, RPW - 1)
            bnd = (dst16 != prev16).astype(F32)
            plsc.store_scatter(pb[par], [e16, jnp.full((16,), 8, jnp.int32)], bnd)
            plsc.store_scatter(pb[par], [e16, jnp.full((16,), 9, jnp.int32)],
                               dst16.astype(F32))
            for h in range(nheads):
                asrc = plsc.load_gather(srcrows, [e16, jnp.full((16,), scol + h, jnp.int32)])
                adst = plsc.load_gather(av_v, [dloc16, jnp.full((16,), acol + h, jnp.int32)])
                a = asrc + adst
                p = jnp.exp(jnp.maximum(a, 0.2 * a))
                plsc.store_scatter(pb[par], [e16, jnp.full((16,), h, jnp.int32)], p)

        cbase = e0a + c * chunk
        lo = jnp.maximum(0, e0 - cbase)
        hi = jnp.minimum(chunk, e1 - cbase)

        def one_edge(e, valid, ecarry):
            d, accs, dss = ecarry
            prow = pb[par][e, pl.ds(0, 16)]
            boundary = jnp.logical_and(prow[8] > 0.5, valid)

            @pl.when(boundary)
            def _():
                finalize(d, accs, dss, jnp.bool_(False))

            d = jnp.where(boundary, prow[9].astype(jnp.int32), d)
            keep = jnp.where(boundary, jnp.float32(0.0), jnp.float32(1.0))
            vf = jnp.where(valid, jnp.float32(1.0), jnp.float32(0.0))

            ps = [prow[h] * vf for h in range(nheads)]
            dss = tuple(dss[h] * keep + ps[h] for h in range(nheads))
            accs = tuple(
                accs[v] * keep + ps[v // vh] * hb[par][e, pl.ds(v * 16, 16)]
                for v in range(nv))
            return d, accs, dss

        def edge_pair(i, ecarry):
            e = lo + i * 2
            ecarry = one_edge(e, jnp.bool_(True), ecarry)
            return one_edge(jnp.minimum(e + 1, chunk - 1), e + 1 < hi, ecarry)

        carry = lax.fori_loop(0, (hi - lo + 1) // 2, edge_pair, carry)

        issue(c + 2, par)
        return carry

    carry0 = (d0, zacc, zds)
    npairs = (nchunks + 1) // 2

    def pair_body(pi, carry):
        for par in range(2):
            carry = process_chunk(pi * 2 + par, par, carry)
        return carry

    d, accs, dss = lax.fori_loop(0, npairs, pair_body, carry0)
    finalize(d, accs, dss, jnp.bool_(True))


def _make_sc_agg(nfeat, nheads, awidth, chunk):
    mesh = plsc.VectorSubcoreMesh(core_axis_name="c", subcore_axis_name="s",
                                  num_cores=2, num_subcores=16)

    @functools.partial(
        pl.kernel,
        out_type=jax.ShapeDtypeStruct((N_PAD, nfeat), F32),
        mesh=mesh,
        compiler_params=pltpu.CompilerParams(needs_layout_passes=False,
                                             use_tc_tiling_on_sc=False),
        scratch_types=dict(
            rp_v=pltpu.VMEM((48,), jnp.int32),
            src_v=pltpu.VMEM((MAXSPAN,), jnp.int32),
            av_v=pltpu.VMEM((RPW, awidth), F32),
            hb=[pltpu.VMEM((chunk, nfeat), F32)] * 2,
            ab=[pltpu.VMEM((chunk, awidth), F32)] * 2,
            db=[pltpu.VMEM((chunk,), jnp.int32)] * 2,
            pv=[pltpu.VMEM((chunk,), jnp.int32)] * 2,
            pb=[pltpu.VMEM((chunk, 16), F32)] * 2,
            obuf=pltpu.VMEM((32, nfeat), F32),
            sh=[pltpu.SemaphoreType.DMA] * 2,
            sa=[pltpu.SemaphoreType.DMA] * 2,
        ),
    )
    def agg(rp_hbm, src_hbm, dst_hbm, prev_hbm, a_hbm, h_hbm, out_hbm, *,
            rp_v, src_v, av_v, hb, ab, db, pv, pb, obuf, sh, sa):
        _sc_agg_body(nfeat, nheads, chunk, rp_hbm, src_hbm, dst_hbm, prev_hbm,
                     a_hbm, h_hbm, out_hbm, rp_v, src_v, av_v, hb, ab, db,
                     pv, pb, obuf, sh, sa)

    return agg


_agg1 = _make_sc_agg(C1, HEADS, 16, 64)
_agg2 = _make_sc_agg(C2, 1, C2, 256)


def kernel(x, edge_index, W1, att_src1, att_dst1, b1, W2, att_src2, att_dst2, b2):
    loop = jnp.arange(N, dtype=jnp.int32)
    src = jnp.concatenate([edge_index[0], loop])
    dst = jnp.concatenate([edge_index[1], loop])
    packed = lax.sort((dst << 14) | src)
    dst_s = packed >> 14
    src_s = packed & jnp.int32(16383)
    rp32 = jnp.searchsorted(dst_s, jnp.arange(0, N_PAD + 1, RPW,
                                              dtype=jnp.int32)).astype(jnp.int32)
    rp32 = jnp.concatenate([rp32, jnp.full((15,), EP, jnp.int32)])
    src_sp = jnp.pad(src_s, (0, SRC_PAD - EP))
    dst_sp = jnp.pad(dst_s, (0, SRC_PAD - EP))
    prev_sp = jnp.concatenate([jnp.full((1,), -1, jnp.int32), dst_sp[:-1]])

    att_s1 = att_src1.reshape(HEADS, HIDDEN)
    att_d1 = att_dst1.reshape(HEADS, HIDDEN)
    eye = jnp.eye(HEADS, dtype=F32)
    A1 = jnp.concatenate(
        [(att_s1[:, :, None] * eye[:, None, :]).reshape(C1, HEADS),
         (att_d1[:, :, None] * eye[:, None, :]).reshape(C1, HEADS)], axis=1)

    xp = jnp.pad(x, ((0, N_PAD - N), (0, 0)))
    h1, a1 = pl.pallas_call(
        _k1_body,
        out_shape=(jax.ShapeDtypeStruct((N_PAD, C1), F32),
                   jax.ShapeDtypeStruct((N_PAD, 16), F32)),
        grid=(NW,),
        in_specs=[
            pl.BlockSpec((RPW, F_IN), lambda i: (i, 0)),
            pl.BlockSpec((F_IN, C1), lambda i: (0, 0)),
            pl.BlockSpec((C1, 16), lambda i: (0, 0)),
        ],
        out_specs=(pl.BlockSpec((RPW, C1), lambda i: (i, 0)),
                   pl.BlockSpec((RPW, 16), lambda i: (i, 0))),
    )(xp, W1, A1)

    out1 = _agg1(rp32, src_sp, dst_sp, prev_sp, a1, h1)

    W2p = jnp.concatenate(
        [W2, W2 @ att_src2.reshape(N_CLASSES, 1), W2 @ att_dst2.reshape(N_CLASSES, 1),
         jnp.zeros((C1, C2 - N_CLASSES - 2), F32)], axis=1)

    h2p = pl.pallas_call(
        _k2_body,
        out_shape=jax.ShapeDtypeStruct((N_PAD, C2), F32),
        grid=(NW,),
        in_specs=[
            pl.BlockSpec((RPW, C1), lambda i: (i, 0)),
            pl.BlockSpec((C1,), lambda i: (0,)),
            pl.BlockSpec((C1, C2), lambda i: (0, 0)),
        ],
        out_specs=pl.BlockSpec((RPW, C2), lambda i: (i, 0)),
    )(out1, b1, W2p)

    out2 = _agg2(rp32, src_sp, dst_sp, prev_sp, h2p, h2p)

    return pl.pallas_call(
        _k3_body,
        out_shape=jax.ShapeDtypeStruct((N, N_CLASSES), F32),
        grid=(10,),
        in_specs=[
            pl.BlockSpec((N // 10, N_CLASSES), lambda i: (i, 0)),
            pl.BlockSpec((N_CLASSES,), lambda i: (0,)),
        ],
        out_specs=pl.BlockSpec((N // 10, N_CLASSES), lambda i: (i, 0)),
    )(out2[:N, :N_CLASSES], b2)
